# baseline (speedup 1.0000x reference)
import functools
import jax
import jax.numpy as jnp
from jax import lax
from jax.experimental import pallas as pl
from jax.experimental.pallas import tpu as pltpu
from jax.experimental.pallas import tpu_sc as plsc

NU = 10000
NI = 10000
E = 160000
DIN = 384
H = 256
BM = 1000

NS = 16
NC = 2
EC = 80
NCH = E // (NS * EC)
KB = 25
NBLK = NCH // KB
NB = 4
ROWS_PER_TILE = NU // NS


def _proj_body(x_ref, w_ref, b_ref, o_ref):
    o_ref[:] = jnp.dot(x_ref[:], w_ref[:], preferred_element_type=jnp.float32) + b_ref[:]


def _proj(x, w, b):
    n, k = x.shape
    h = w.shape[1]
    return pl.pallas_call(
        _proj_body,
        grid=(n // BM,),
        in_specs=[
            pl.BlockSpec((BM, k), lambda m: (m, 0)),
            pl.BlockSpec((k, h), lambda m: (0, 0)),
            pl.BlockSpec((1, h), lambda m: (0, 0)),
        ],
        out_specs=pl.BlockSpec((BM, h), lambda m: (m, 0)),
        out_shape=jax.ShapeDtypeStruct((n, h), jnp.float32),
    )(x, w, b.reshape(1, h))


def _left_body(h_ref, w_ref, o_ref):
    o_ref[:] = jnp.dot(h_ref[:], w_ref[:], preferred_element_type=jnp.float32)


def _left_proj(h, wl):
    n = h.shape[0]
    nm = n // BM
    return pl.pallas_call(
        _left_body,
        grid=(2, nm),
        in_specs=[
            pl.BlockSpec((BM, H), lambda c, m: (m, 0)),
            pl.BlockSpec((H, 128), lambda c, m: (0, c)),
        ],
        out_specs=pl.BlockSpec((BM, 128), lambda c, m: (c * nm + m, 0)),
        out_shape=jax.ShapeDtypeStruct((2 * n, 128), jnp.float32),
    )(h, wl)


def _epi_body(s_ref, inv_ref, h_ref, w_ref, b_ref, o_ref):
    right = jnp.dot(h_ref[:], w_ref[:], preferred_element_type=jnp.float32) + b_ref[:]
    left = jnp.concatenate([s_ref[0], s_ref[1]], axis=1) * inv_ref[:]
    o_ref[:] = jnp.maximum(left + right, 0.0)


def _epilogue(s2, inv, h, wr, b):
    n = h.shape[0]
    return pl.pallas_call(
        _epi_body,
        grid=(n // BM,),
        in_specs=[
            pl.BlockSpec((2, BM, 128), lambda m: (0, m, 0)),
            pl.BlockSpec((BM, 1), lambda m: (m, 0)),
            pl.BlockSpec((BM, H), lambda m: (m, 0)),
            pl.BlockSpec((H, H), lambda m: (0, 0)),
            pl.BlockSpec((1, H), lambda m: (0, 0)),
        ],
        out_specs=pl.BlockSpec((BM, H), lambda m: (m, 0)),
        out_shape=jax.ShapeDtypeStruct((n, H), jnp.float32),
    )(s2, inv, h, wr, b.reshape(1, H))


_MESH = plsc.VectorSubcoreMesh(core_axis_name="c", subcore_axis_name="s")


def _segsum_body(p2, srcs, dsts, zeros, out, acc, srcv, dstv, rows, gsem, ssem):
    c = lax.axis_index("c")
    s = lax.axis_index("s")
    @pl.when(s < 10)
    def _():
        pltpu.sync_copy(zeros, acc.at[pl.ds(s * 1000, 1000)])
    plsc.subcore_barrier()

    def block(kb, carry):
        pltpu.sync_copy(srcs.at[c, s, kb], srcv)
        pltpu.sync_copy(dsts.at[s, kb], dstv)

        def step(t, carry2):
            b = lax.rem(t, NB)
            @pl.when(t >= NB)
            def _():
                pltpu.make_async_copy(rows.at[b], acc.at[dstv.at[t]],
                                      ssem.at[b]).wait()
            pltpu.async_copy(p2.at[srcv.at[t]], rows.at[b], gsem).wait()
            pltpu.async_copy(rows.at[b], acc.at[dstv.at[t]], ssem.at[b],
                             add=True)
            return carry2

        lax.fori_loop(0, KB, step, 0)
        for b in range(NB):
            pltpu.make_async_copy(rows.at[b], acc.at[dstv.at[b]],
                                  ssem.at[b]).wait()
        return carry

    lax.fori_loop(0, NBLK, block, 0)
    plsc.subcore_barrier()
    @pl.when(s < 10)
    def _():
        pltpu.sync_copy(acc.at[pl.ds(s * 1000, 1000)],
                        out.at[pl.ds(c * NU + s * 1000, 1000)])


_segsum_call = pl.kernel(
    _segsum_body,
    out_type=jax.ShapeDtypeStruct((2 * NU, 128), jnp.float32),
    mesh=_MESH,
    scratch_types=[
        pltpu.VMEM_SHARED((NU, 128), jnp.float32),
        pltpu.VMEM((KB, EC), jnp.int32),
        pltpu.VMEM((KB, EC), jnp.int32),
        pltpu.VMEM((NB, EC, 128), jnp.float32),
        pltpu.SemaphoreType.DMA,
        pltpu.SemaphoreType.DMA((NB,)),
    ],
)


def _segsum(p2, srcs3, dsts3, zeros):
    return _segsum_call(p2, srcs3, dsts3, zeros).reshape(2, NU, 128)


_EPT = E // NS


def _counts_body(dsts, ones_hbm, zeros, out, acc, dstv, ones_buf):
    c = lax.axis_index("c")
    s = lax.axis_index("s")

    @pl.when(s < 10)
    def _():
        pltpu.sync_copy(zeros, acc.at[pl.ds(s * 1000, 1000)])
    pltpu.sync_copy(ones_hbm, ones_buf)
    pltpu.sync_copy(dsts.at[c, s], dstv)
    plsc.subcore_barrier()

    def step(t, carry):
        pltpu.sync_copy(ones_buf, acc.at[dstv.at[t // KB, t % KB]], add=True)
        return carry

    lax.fori_loop(0, NCH, step, 0)
    plsc.subcore_barrier()
    @pl.when(s < 10)
    def _():
        pltpu.sync_copy(acc.at[pl.ds(s * 1000, 1000)], out.at[c, pl.ds(s * 1000, 1000)])


_counts_call = pl.kernel(
    _counts_body,
    out_type=jax.ShapeDtypeStruct((2, NU, 128), jnp.float32),
    mesh=_MESH,
    scratch_types=[
        pltpu.VMEM_SHARED((NU, 128), jnp.float32),
        pltpu.VMEM((NBLK, KB, EC), jnp.int32),
        pltpu.VMEM((EC, 128), jnp.float32),
    ],
)


def kernel(x_user, x_item, ei_u2i, ei_i2u, Win_u, bin_u, Win_i, bin_i, Wl0_u2i, bl0_u2i, Wr0_u2i, br0_u2i, Wl0_i2u, bl0_i2u, Wr0_i2u, br0_i2u, Wl1_u2i, bl1_u2i, Wr1_u2i, br1_u2i, Wl1_i2u, bl1_i2u, Wr1_i2u, br1_i2u):
    def prep(ei, n_src):
        src = ei[0].reshape(NS, NBLK, KB, EC)
        srcs3 = jnp.stack([src, src + n_src])
        dsts3 = ei[1].reshape(NS, NBLK, KB, EC)
        return srcs3, dsts3

    srcs_u2i, dsts_u2i = prep(ei_u2i, NU)
    srcs_i2u, dsts_i2u = prep(ei_i2u, NI)
    zeros = jnp.zeros((1000, 128), jnp.float32)

    dst_both = jnp.stack([dsts_u2i, dsts_i2u])
    ones_p = jnp.ones((EC, 128), jnp.float32)
    cnts = _counts_call(dst_both, ones_p, zeros)
    inv_i = (1.0 / jnp.maximum(cnts[0, :, 0], 1.0)).reshape(NI, 1)
    inv_u = (1.0 / jnp.maximum(cnts[1, :, 0], 1.0)).reshape(NU, 1)

    hu = _proj(x_user, Win_u, bin_u)
    hi = _proj(x_item, Win_i, bin_i)

    layers = [
        (Wl0_u2i, bl0_u2i, Wr0_u2i, br0_u2i, Wl0_i2u, bl0_i2u, Wr0_i2u, br0_i2u),
        (Wl1_u2i, bl1_u2i, Wr1_u2i, br1_u2i, Wl1_i2u, bl1_i2u, Wr1_i2u, br1_i2u),
    ]
    for (Wlu2i, blu2i, Wru2i, bru2i, Wli2u, bli2u, Wri2u, bri2u) in layers:
        pi = _left_proj(hu, Wlu2i)
        si = _segsum(pi, srcs_u2i, dsts_u2i, zeros)
        pu = _left_proj(hi, Wli2u)
        su = _segsum(pu, srcs_i2u, dsts_i2u, zeros)
        new_i = _epilogue(si, inv_i, hi, Wru2i, blu2i + bru2i)
        new_u = _epilogue(su, inv_u, hu, Wri2u, bli2u + bri2u)
        hu, hi = new_u, new_i
    return hu, hi


# speedup vs baseline: 1.2143x; 1.2143x over previous
import functools
import jax
import jax.numpy as jnp
from jax import lax
from jax.experimental import pallas as pl
from jax.experimental.pallas import tpu as pltpu
from jax.experimental.pallas import tpu_sc as plsc

NU = 10000
NI = 10000
E = 160000
DIN = 384
H = 256
BM = 1000

NS = 16
NC = 2
EC = 125
NCH = E // (NS * EC)
KB = 16
NBLK = NCH // KB
GRP = 4
ROWS_PER_TILE = NU // NS


def _proj_body(x_ref, w_ref, b_ref, o_ref):
    o_ref[:] = jnp.dot(x_ref[:], w_ref[:], preferred_element_type=jnp.float32) + b_ref[:]


def _proj(x, w, b):
    n, k = x.shape
    h = w.shape[1]
    return pl.pallas_call(
        _proj_body,
        grid=(n // BM,),
        in_specs=[
            pl.BlockSpec((BM, k), lambda m: (m, 0)),
            pl.BlockSpec((k, h), lambda m: (0, 0)),
            pl.BlockSpec((1, h), lambda m: (0, 0)),
        ],
        out_specs=pl.BlockSpec((BM, h), lambda m: (m, 0)),
        out_shape=jax.ShapeDtypeStruct((n, h), jnp.float32),
    )(x, w, b.reshape(1, h))


def _left_body(h_ref, w_ref, o_ref):
    o_ref[:] = jnp.dot(h_ref[:], w_ref[:], preferred_element_type=jnp.float32)


def _left_proj(h, wl):
    n = h.shape[0]
    nm = n // BM
    return pl.pallas_call(
        _left_body,
        grid=(2, nm),
        in_specs=[
            pl.BlockSpec((BM, H), lambda c, m: (m, 0)),
            pl.BlockSpec((H, 128), lambda c, m: (0, c)),
        ],
        out_specs=pl.BlockSpec((BM, 128), lambda c, m: (c * nm + m, 0)),
        out_shape=jax.ShapeDtypeStruct((2 * n, 128), jnp.float32),
    )(h, wl)


def _epi_body(s_ref, inv_ref, h_ref, w_ref, b_ref, o_ref):
    right = jnp.dot(h_ref[:], w_ref[:], preferred_element_type=jnp.float32) + b_ref[:]
    left = jnp.concatenate([s_ref[0], s_ref[1]], axis=1) * inv_ref[:]
    o_ref[:] = jnp.maximum(left + right, 0.0)


def _epilogue(s2, inv, h, wr, b):
    n = h.shape[0]
    return pl.pallas_call(
        _epi_body,
        grid=(n // BM,),
        in_specs=[
            pl.BlockSpec((2, BM, 128), lambda m: (0, m, 0)),
            pl.BlockSpec((BM, 1), lambda m: (m, 0)),
            pl.BlockSpec((BM, H), lambda m: (m, 0)),
            pl.BlockSpec((H, H), lambda m: (0, 0)),
            pl.BlockSpec((1, H), lambda m: (0, 0)),
        ],
        out_specs=pl.BlockSpec((BM, H), lambda m: (m, 0)),
        out_shape=jax.ShapeDtypeStruct((n, H), jnp.float32),
    )(s2, inv, h, wr, b.reshape(1, H))


_MESH = plsc.VectorSubcoreMesh(core_axis_name="c", subcore_axis_name="s")


def _segsum_body(p2, srcs, dsts, zeros, out, acc, srcv, dstv, rows, gsem, ssem):
    c = lax.axis_index("c")
    s = lax.axis_index("s")
    @pl.when(s < 10)
    def _():
        pltpu.sync_copy(zeros, acc.at[pl.ds(s * 1000, 1000)])
    plsc.subcore_barrier()

    def block(kb, carry):
        pltpu.sync_copy(srcs.at[c, s, kb], srcv)
        pltpu.sync_copy(dsts.at[s, kb], dstv)

        def group(g, carry2):
            base = g * GRP
            g0 = pltpu.async_copy(p2.at[srcv.at[base]], rows.at[0], gsem)
            g1 = pltpu.async_copy(p2.at[srcv.at[base + 1]], rows.at[1], gsem)
            g0.wait()
            s0 = pltpu.async_copy(rows.at[0], acc.at[dstv.at[base]],
                                  ssem.at[0], add=True)
            g1.wait()
            s1 = pltpu.async_copy(rows.at[1], acc.at[dstv.at[base + 1]],
                                  ssem.at[1], add=True)
            s0.wait()
            g2 = pltpu.async_copy(p2.at[srcv.at[base + 2]], rows.at[0], gsem)
            s1.wait()
            g3 = pltpu.async_copy(p2.at[srcv.at[base + 3]], rows.at[1], gsem)
            g2.wait()
            s2 = pltpu.async_copy(rows.at[0], acc.at[dstv.at[base + 2]],
                                  ssem.at[0], add=True)
            g3.wait()
            s3 = pltpu.async_copy(rows.at[1], acc.at[dstv.at[base + 3]],
                                  ssem.at[1], add=True)
            s2.wait()
            s3.wait()
            return carry2

        lax.fori_loop(0, KB // GRP, group, 0)
        return carry

    lax.fori_loop(0, NBLK, block, 0)
    plsc.subcore_barrier()
    @pl.when(s < 10)
    def _():
        pltpu.sync_copy(acc.at[pl.ds(s * 1000, 1000)],
                        out.at[pl.ds(c * NU + s * 1000, 1000)])


_segsum_call = pl.kernel(
    _segsum_body,
    out_type=jax.ShapeDtypeStruct((2 * NU, 128), jnp.float32),
    mesh=_MESH,
    scratch_types=[
        pltpu.VMEM_SHARED((NU, 128), jnp.float32),
        pltpu.VMEM((KB, EC), jnp.int32),
        pltpu.VMEM((KB, EC), jnp.int32),
        pltpu.VMEM((2, EC, 128), jnp.float32),
        pltpu.SemaphoreType.DMA,
        pltpu.SemaphoreType.DMA((2,)),
    ],
)


def _segsum(p2, srcs3, dsts3, zeros):
    return _segsum_call(p2, srcs3, dsts3, zeros).reshape(2, NU, 128)


_EPT = E // NS


def _counts_body(dsts, ones_hbm, zeros, out, acc, dstv, ones_buf):
    c = lax.axis_index("c")
    s = lax.axis_index("s")

    @pl.when(s < 10)
    def _():
        pltpu.sync_copy(zeros, acc.at[pl.ds(s * 1000, 1000)])
    pltpu.sync_copy(ones_hbm, ones_buf)
    pltpu.sync_copy(dsts.at[c, s], dstv)
    plsc.subcore_barrier()

    def step(t, carry):
        pltpu.sync_copy(ones_buf, acc.at[dstv.at[t // KB, t % KB]], add=True)
        return carry

    lax.fori_loop(0, NCH, step, 0)
    plsc.subcore_barrier()
    @pl.when(s < 10)
    def _():
        pltpu.sync_copy(acc.at[pl.ds(s * 1000, 1000)], out.at[c, pl.ds(s * 1000, 1000)])


_counts_call = pl.kernel(
    _counts_body,
    out_type=jax.ShapeDtypeStruct((2, NU, 128), jnp.float32),
    mesh=_MESH,
    scratch_types=[
        pltpu.VMEM_SHARED((NU, 128), jnp.float32),
        pltpu.VMEM((NBLK, KB, EC), jnp.int32),
        pltpu.VMEM((EC, 128), jnp.float32),
    ],
)


def kernel(x_user, x_item, ei_u2i, ei_i2u, Win_u, bin_u, Win_i, bin_i, Wl0_u2i, bl0_u2i, Wr0_u2i, br0_u2i, Wl0_i2u, bl0_i2u, Wr0_i2u, br0_i2u, Wl1_u2i, bl1_u2i, Wr1_u2i, br1_u2i, Wl1_i2u, bl1_i2u, Wr1_i2u, br1_i2u):
    def prep(ei, n_src):
        src = ei[0].reshape(NS, NBLK, KB, EC)
        srcs3 = jnp.stack([src, src + n_src])
        dsts3 = ei[1].reshape(NS, NBLK, KB, EC)
        return srcs3, dsts3

    srcs_u2i, dsts_u2i = prep(ei_u2i, NU)
    srcs_i2u, dsts_i2u = prep(ei_i2u, NI)
    zeros = jnp.zeros((1000, 128), jnp.float32)

    dst_both = jnp.stack([dsts_u2i, dsts_i2u])
    ones_p = jnp.ones((EC, 128), jnp.float32)
    cnts = _counts_call(dst_both, ones_p, zeros)
    inv_i = (1.0 / jnp.maximum(cnts[0, :, 0], 1.0)).reshape(NI, 1)
    inv_u = (1.0 / jnp.maximum(cnts[1, :, 0], 1.0)).reshape(NU, 1)

    hu = _proj(x_user, Win_u, bin_u)
    hi = _proj(x_item, Win_i, bin_i)

    layers = [
        (Wl0_u2i, bl0_u2i, Wr0_u2i, br0_u2i, Wl0_i2u, bl0_i2u, Wr0_i2u, br0_i2u),
        (Wl1_u2i, bl1_u2i, Wr1_u2i, br1_u2i, Wl1_i2u, bl1_i2u, Wr1_i2u, br1_i2u),
    ]
    for (Wlu2i, blu2i, Wru2i, bru2i, Wli2u, bli2u, Wri2u, bri2u) in layers:
        pi = _left_proj(hu, Wlu2i)
        si = _segsum(pi, srcs_u2i, dsts_u2i, zeros)
        pu = _left_proj(hi, Wli2u)
        su = _segsum(pu, srcs_i2u, dsts_i2u, zeros)
        new_i = _epilogue(si, inv_i, hi, Wru2i, blu2i + bru2i)
        new_u = _epilogue(su, inv_u, hu, Wri2u, bli2u + bri2u)
        hu, hi = new_u, new_i
    return hu, hi


# speedup vs baseline: 1.2154x; 1.0009x over previous
import functools
import jax
import jax.numpy as jnp
from jax import lax
from jax.experimental import pallas as pl
from jax.experimental.pallas import tpu as pltpu
from jax.experimental.pallas import tpu_sc as plsc

NU = 10000
NI = 10000
E = 160000
DIN = 384
H = 256
BM = 1000

NS = 16
NC = 2
EC = 125
NCH = E // (NS * EC)
KB = 16
NBLK = NCH // KB
GRP = 4
ROWS_PER_TILE = NU // NS


def _proj_body(x_ref, w_ref, b_ref, o_ref):
    o_ref[:] = jnp.dot(x_ref[:], w_ref[:], preferred_element_type=jnp.float32) + b_ref[:]


def _proj(x, w, b):
    n, k = x.shape
    h = w.shape[1]
    return pl.pallas_call(
        _proj_body,
        grid=(n // BM,),
        in_specs=[
            pl.BlockSpec((BM, k), lambda m: (m, 0)),
            pl.BlockSpec((k, h), lambda m: (0, 0)),
            pl.BlockSpec((1, h), lambda m: (0, 0)),
        ],
        out_specs=pl.BlockSpec((BM, h), lambda m: (m, 0)),
        out_shape=jax.ShapeDtypeStruct((n, h), jnp.float32),
    )(x, w, b.reshape(1, h))


def _left_body(h_ref, w_ref, o_ref):
    o_ref[:] = jnp.dot(h_ref[:], w_ref[:], preferred_element_type=jnp.float32)


def _left_proj(h, wl):
    n = h.shape[0]
    nm = n // BM
    return pl.pallas_call(
        _left_body,
        grid=(2, nm),
        in_specs=[
            pl.BlockSpec((BM, H), lambda c, m: (m, 0)),
            pl.BlockSpec((H, 128), lambda c, m: (0, c)),
        ],
        out_specs=pl.BlockSpec((BM, 128), lambda c, m: (c * nm + m, 0)),
        out_shape=jax.ShapeDtypeStruct((2 * n, 128), jnp.float32),
    )(h, wl)


def _epi_body(s_ref, inv_ref, h_ref, w_ref, b_ref, o_ref):
    right = jnp.dot(h_ref[:], w_ref[:], preferred_element_type=jnp.float32) + b_ref[:]
    left = jnp.concatenate([s_ref[0], s_ref[1]], axis=1) * inv_ref[:]
    o_ref[:] = jnp.maximum(left + right, 0.0)


def _epilogue(s2, inv, h, wr, b):
    n = h.shape[0]
    return pl.pallas_call(
        _epi_body,
        grid=(n // BM,),
        in_specs=[
            pl.BlockSpec((2, BM, 128), lambda m: (0, m, 0)),
            pl.BlockSpec((BM, 1), lambda m: (m, 0)),
            pl.BlockSpec((BM, H), lambda m: (m, 0)),
            pl.BlockSpec((H, H), lambda m: (0, 0)),
            pl.BlockSpec((1, H), lambda m: (0, 0)),
        ],
        out_specs=pl.BlockSpec((BM, H), lambda m: (m, 0)),
        out_shape=jax.ShapeDtypeStruct((n, H), jnp.float32),
    )(s2, inv, h, wr, b.reshape(1, H))


_MESH = plsc.VectorSubcoreMesh(core_axis_name="c", subcore_axis_name="s")


def _segsum_body(p2, srcs, dsts, zeros, out, acc, srcv, dstv, rows, gsem, ssem):
    c = lax.axis_index("c")
    s = lax.axis_index("s")
    @pl.when(s < 10)
    def _():
        pltpu.sync_copy(zeros, acc.at[pl.ds(s * 1000, 1000)])
    plsc.subcore_barrier()

    def block(kb, carry):
        pltpu.sync_copy(srcs.at[c, s, kb], srcv)
        pltpu.sync_copy(dsts.at[s, kb], dstv)

        def group(g, carry2):
            base = g * GRP
            g0 = pltpu.async_copy(p2.at[srcv.at[base]], rows.at[0], gsem)
            g1 = pltpu.async_copy(p2.at[srcv.at[base + 1]], rows.at[1], gsem)
            g0.wait()
            s0 = pltpu.async_copy(rows.at[0], acc.at[dstv.at[base]],
                                  ssem.at[0], add=True)
            g1.wait()
            s1 = pltpu.async_copy(rows.at[1], acc.at[dstv.at[base + 1]],
                                  ssem.at[1], add=True)
            s0.wait()
            g2 = pltpu.async_copy(p2.at[srcv.at[base + 2]], rows.at[0], gsem)
            s1.wait()
            g3 = pltpu.async_copy(p2.at[srcv.at[base + 3]], rows.at[1], gsem)
            g2.wait()
            s2 = pltpu.async_copy(rows.at[0], acc.at[dstv.at[base + 2]],
                                  ssem.at[0], add=True)
            g3.wait()
            s3 = pltpu.async_copy(rows.at[1], acc.at[dstv.at[base + 3]],
                                  ssem.at[1], add=True)
            s2.wait()
            s3.wait()
            return carry2

        lax.fori_loop(0, KB // GRP, group, 0)
        return carry

    lax.fori_loop(0, NBLK, block, 0)
    plsc.subcore_barrier()
    @pl.when(s < 10)
    def _():
        pltpu.sync_copy(acc.at[pl.ds(s * 1000, 1000)],
                        out.at[pl.ds(c * NU + s * 1000, 1000)])


_segsum_call = pl.kernel(
    _segsum_body,
    out_type=jax.ShapeDtypeStruct((2 * NU, 128), jnp.float32),
    mesh=_MESH,
    scratch_types=[
        pltpu.VMEM_SHARED((NU, 128), jnp.float32),
        pltpu.VMEM((KB, EC), jnp.int32),
        pltpu.VMEM((KB, EC), jnp.int32),
        pltpu.VMEM((2, EC, 128), jnp.float32),
        pltpu.SemaphoreType.DMA,
        pltpu.SemaphoreType.DMA((2,)),
    ],
)


def _segsum(p2, srcs3, dsts3, zeros):
    return _segsum_call(p2, srcs3, dsts3, zeros).reshape(2, NU, 128)


_EPT = E // NS


def _counts_body(dsts, ones_hbm, zeros, out, acc, dstv, ones_buf, csem):
    c = lax.axis_index("c")
    s = lax.axis_index("s")

    @pl.when(s < 10)
    def _():
        pltpu.sync_copy(zeros, acc.at[pl.ds(s * 1000, 1000)])
    pltpu.sync_copy(ones_hbm, ones_buf)
    pltpu.sync_copy(dsts.at[c, s], dstv)
    plsc.subcore_barrier()

    def group(g, carry):
        base = g * GRP
        descs = []
        for i in range(GRP):
            t = base + i
            descs.append(pltpu.async_copy(
                ones_buf, acc.at[dstv.at[t // KB, t % KB]], csem.at[i],
                add=True))
        for d in descs:
            d.wait()
        return carry

    lax.fori_loop(0, NCH // GRP, group, 0)
    plsc.subcore_barrier()
    @pl.when(s < 10)
    def _():
        pltpu.sync_copy(acc.at[pl.ds(s * 1000, 1000)], out.at[c, pl.ds(s * 1000, 1000)])


_counts_call = pl.kernel(
    _counts_body,
    out_type=jax.ShapeDtypeStruct((2, NU, 128), jnp.float32),
    mesh=_MESH,
    scratch_types=[
        pltpu.VMEM_SHARED((NU, 128), jnp.float32),
        pltpu.VMEM((NBLK, KB, EC), jnp.int32),
        pltpu.VMEM((EC, 128), jnp.float32),
        pltpu.SemaphoreType.DMA((GRP,)),
    ],
)


def kernel(x_user, x_item, ei_u2i, ei_i2u, Win_u, bin_u, Win_i, bin_i, Wl0_u2i, bl0_u2i, Wr0_u2i, br0_u2i, Wl0_i2u, bl0_i2u, Wr0_i2u, br0_i2u, Wl1_u2i, bl1_u2i, Wr1_u2i, br1_u2i, Wl1_i2u, bl1_i2u, Wr1_i2u, br1_i2u):
    def prep(ei, n_src):
        src = ei[0].reshape(NS, NBLK, KB, EC)
        srcs3 = jnp.stack([src, src + n_src])
        dsts3 = ei[1].reshape(NS, NBLK, KB, EC)
        return srcs3, dsts3

    srcs_u2i, dsts_u2i = prep(ei_u2i, NU)
    srcs_i2u, dsts_i2u = prep(ei_i2u, NI)
    zeros = jnp.zeros((1000, 128), jnp.float32)

    dst_both = jnp.stack([dsts_u2i, dsts_i2u])
    ones_p = jnp.ones((EC, 128), jnp.float32)
    cnts = _counts_call(dst_both, ones_p, zeros)
    inv_i = (1.0 / jnp.maximum(cnts[0, :, 0], 1.0)).reshape(NI, 1)
    inv_u = (1.0 / jnp.maximum(cnts[1, :, 0], 1.0)).reshape(NU, 1)

    hu = _proj(x_user, Win_u, bin_u)
    hi = _proj(x_item, Win_i, bin_i)

    layers = [
        (Wl0_u2i, bl0_u2i, Wr0_u2i, br0_u2i, Wl0_i2u, bl0_i2u, Wr0_i2u, br0_i2u),
        (Wl1_u2i, bl1_u2i, Wr1_u2i, br1_u2i, Wl1_i2u, bl1_i2u, Wr1_i2u, br1_i2u),
    ]
    for (Wlu2i, blu2i, Wru2i, bru2i, Wli2u, bli2u, Wri2u, bri2u) in layers:
        pi = _left_proj(hu, Wlu2i)
        si = _segsum(pi, srcs_u2i, dsts_u2i, zeros)
        pu = _left_proj(hi, Wli2u)
        su = _segsum(pu, srcs_i2u, dsts_i2u, zeros)
        new_i = _epilogue(si, inv_i, hi, Wru2i, blu2i + bru2i)
        new_u = _epilogue(su, inv_u, hu, Wri2u, bli2u + bri2u)
        hu, hi = new_u, new_i
    return hu, hi


# speedup vs baseline: 1.2252x; 1.0081x over previous
import functools
import jax
import jax.numpy as jnp
from jax import lax
from jax.experimental import pallas as pl
from jax.experimental.pallas import tpu as pltpu
from jax.experimental.pallas import tpu_sc as plsc

NU = 10000
NI = 10000
E = 160000
DIN = 384
H = 256
BM = 1000

NS = 16
NC = 2
EC = 125
NCH = E // (NS * EC)
KB = 16
NBLK = NCH // KB
GRP = 4
ROWS_PER_TILE = NU // NS


def _proj_body(x_ref, w_ref, b_ref, o_ref):
    o_ref[:] = jnp.dot(x_ref[:], w_ref[:], preferred_element_type=jnp.float32) + b_ref[:]


def _proj(x, w, b):
    n, k = x.shape
    h = w.shape[1]
    return pl.pallas_call(
        _proj_body,
        grid=(n // BM,),
        in_specs=[
            pl.BlockSpec((BM, k), lambda m: (m, 0)),
            pl.BlockSpec((k, h), lambda m: (0, 0)),
            pl.BlockSpec((1, h), lambda m: (0, 0)),
        ],
        out_specs=pl.BlockSpec((BM, h), lambda m: (m, 0)),
        out_shape=jax.ShapeDtypeStruct((n, h), jnp.float32),
    )(x, w, b.reshape(1, h))


def _split(p):
    return jnp.stack([p[:, :128], p[:, 128:]])


def _proj_left_body(x_ref, w_ref, b_ref, wl_ref, h_ref, p_ref):
    h = jnp.dot(x_ref[:], w_ref[:], preferred_element_type=jnp.float32) + b_ref[:]
    h_ref[:] = h
    p_ref[:] = _split(jnp.dot(h, wl_ref[:], preferred_element_type=jnp.float32))


def _proj_left(x, w, b, wl):
    n, k = x.shape
    return pl.pallas_call(
        _proj_left_body,
        grid=(n // BM,),
        in_specs=[
            pl.BlockSpec((BM, k), lambda m: (m, 0)),
            pl.BlockSpec((k, H), lambda m: (0, 0)),
            pl.BlockSpec((1, H), lambda m: (0, 0)),
            pl.BlockSpec((H, H), lambda m: (0, 0)),
        ],
        out_specs=[
            pl.BlockSpec((BM, H), lambda m: (m, 0)),
            pl.BlockSpec((2, BM, 128), lambda m: (0, m, 0)),
        ],
        out_shape=[
            jax.ShapeDtypeStruct((n, H), jnp.float32),
            jax.ShapeDtypeStruct((2, n, 128), jnp.float32),
        ],
    )(x, w, b.reshape(1, H), wl)


def _epi_body(s_ref, inv_ref, h_ref, w_ref, b_ref, o_ref):
    right = jnp.dot(h_ref[:], w_ref[:], preferred_element_type=jnp.float32) + b_ref[:]
    left = jnp.concatenate([s_ref[0], s_ref[1]], axis=1) * inv_ref[:]
    o_ref[:] = jnp.maximum(left + right, 0.0)


def _epi_left_body(s_ref, inv_ref, h_ref, w_ref, b_ref, wl_ref, o_ref, p_ref):
    right = jnp.dot(h_ref[:], w_ref[:], preferred_element_type=jnp.float32) + b_ref[:]
    left = jnp.concatenate([s_ref[0], s_ref[1]], axis=1) * inv_ref[:]
    o = jnp.maximum(left + right, 0.0)
    o_ref[:] = o
    p_ref[:] = _split(jnp.dot(o, wl_ref[:], preferred_element_type=jnp.float32))


def _epi_left(s2, inv, h, wr, b, wl):
    n = h.shape[0]
    return pl.pallas_call(
        _epi_left_body,
        grid=(n // BM,),
        in_specs=[
            pl.BlockSpec((2, BM, 128), lambda m: (0, m, 0)),
            pl.BlockSpec((BM, 1), lambda m: (m, 0)),
            pl.BlockSpec((BM, H), lambda m: (m, 0)),
            pl.BlockSpec((H, H), lambda m: (0, 0)),
            pl.BlockSpec((1, H), lambda m: (0, 0)),
            pl.BlockSpec((H, H), lambda m: (0, 0)),
        ],
        out_specs=[
            pl.BlockSpec((BM, H), lambda m: (m, 0)),
            pl.BlockSpec((2, BM, 128), lambda m: (0, m, 0)),
        ],
        out_shape=[
            jax.ShapeDtypeStruct((n, H), jnp.float32),
            jax.ShapeDtypeStruct((2, n, 128), jnp.float32),
        ],
    )(s2, inv, h, wr, b.reshape(1, H), wl)


def _epilogue(s2, inv, h, wr, b):
    n = h.shape[0]
    return pl.pallas_call(
        _epi_body,
        grid=(n // BM,),
        in_specs=[
            pl.BlockSpec((2, BM, 128), lambda m: (0, m, 0)),
            pl.BlockSpec((BM, 1), lambda m: (m, 0)),
            pl.BlockSpec((BM, H), lambda m: (m, 0)),
            pl.BlockSpec((H, H), lambda m: (0, 0)),
            pl.BlockSpec((1, H), lambda m: (0, 0)),
        ],
        out_specs=pl.BlockSpec((BM, H), lambda m: (m, 0)),
        out_shape=jax.ShapeDtypeStruct((n, H), jnp.float32),
    )(s2, inv, h, wr, b.reshape(1, H))


_MESH = plsc.VectorSubcoreMesh(core_axis_name="c", subcore_axis_name="s")


def _segsum_body(p2, srcs, dsts, zeros, out, acc, srcv, dstv, rows, gsem, ssem):
    c = lax.axis_index("c")
    s = lax.axis_index("s")
    @pl.when(s < 10)
    def _():
        pltpu.sync_copy(zeros, acc.at[pl.ds(s * 1000, 1000)])
    plsc.subcore_barrier()

    def block(kb, carry):
        pltpu.sync_copy(srcs.at[c, s, kb], srcv)
        pltpu.sync_copy(dsts.at[s, kb], dstv)

        def group(g, carry2):
            base = g * GRP
            g0 = pltpu.async_copy(p2.at[srcv.at[base]], rows.at[0], gsem)
            g1 = pltpu.async_copy(p2.at[srcv.at[base + 1]], rows.at[1], gsem)
            g0.wait()
            s0 = pltpu.async_copy(rows.at[0], acc.at[dstv.at[base]],
                                  ssem.at[0], add=True)
            g1.wait()
            s1 = pltpu.async_copy(rows.at[1], acc.at[dstv.at[base + 1]],
                                  ssem.at[1], add=True)
            s0.wait()
            g2 = pltpu.async_copy(p2.at[srcv.at[base + 2]], rows.at[0], gsem)
            s1.wait()
            g3 = pltpu.async_copy(p2.at[srcv.at[base + 3]], rows.at[1], gsem)
            g2.wait()
            s2 = pltpu.async_copy(rows.at[0], acc.at[dstv.at[base + 2]],
                                  ssem.at[0], add=True)
            g3.wait()
            s3 = pltpu.async_copy(rows.at[1], acc.at[dstv.at[base + 3]],
                                  ssem.at[1], add=True)
            s2.wait()
            s3.wait()
            return carry2

        lax.fori_loop(0, KB // GRP, group, 0)
        return carry

    lax.fori_loop(0, NBLK, block, 0)
    plsc.subcore_barrier()
    @pl.when(s < 10)
    def _():
        pltpu.sync_copy(acc.at[pl.ds(s * 1000, 1000)],
                        out.at[pl.ds(c * NU + s * 1000, 1000)])


_segsum_call = pl.kernel(
    _segsum_body,
    out_type=jax.ShapeDtypeStruct((2 * NU, 128), jnp.float32),
    mesh=_MESH,
    scratch_types=[
        pltpu.VMEM_SHARED((NU, 128), jnp.float32),
        pltpu.VMEM((KB, EC), jnp.int32),
        pltpu.VMEM((KB, EC), jnp.int32),
        pltpu.VMEM((2, EC, 128), jnp.float32),
        pltpu.SemaphoreType.DMA,
        pltpu.SemaphoreType.DMA((2,)),
    ],
)


def _segsum(p2, srcs3, dsts3, zeros):
    return _segsum_call(p2, srcs3, dsts3, zeros).reshape(2, NU, 128)


_EPT = E // NS


def _counts_body(dsts, ones_hbm, zeros, out, acc, dstv, ones_buf, csem):
    c = lax.axis_index("c")
    s = lax.axis_index("s")

    @pl.when(s < 10)
    def _():
        pltpu.sync_copy(zeros, acc.at[pl.ds(s * 1000, 1000)])
    pltpu.sync_copy(ones_hbm, ones_buf)
    pltpu.sync_copy(dsts.at[c, s], dstv)
    plsc.subcore_barrier()

    def group(g, carry):
        base = g * GRP
        descs = []
        for i in range(GRP):
            t = base + i
            descs.append(pltpu.async_copy(
                ones_buf, acc.at[dstv.at[t // KB, t % KB]], csem.at[i],
                add=True))
        for d in descs:
            d.wait()
        return carry

    lax.fori_loop(0, NCH // GRP, group, 0)
    plsc.subcore_barrier()
    @pl.when(s < 10)
    def _():
        pltpu.sync_copy(acc.at[pl.ds(s * 1000, 1000)], out.at[c, pl.ds(s * 1000, 1000)])


_counts_call = pl.kernel(
    _counts_body,
    out_type=jax.ShapeDtypeStruct((2, NU, 128), jnp.float32),
    mesh=_MESH,
    scratch_types=[
        pltpu.VMEM_SHARED((NU, 128), jnp.float32),
        pltpu.VMEM((NBLK, KB, EC), jnp.int32),
        pltpu.VMEM((EC, 128), jnp.float32),
        pltpu.SemaphoreType.DMA((GRP,)),
    ],
)


def kernel(x_user, x_item, ei_u2i, ei_i2u, Win_u, bin_u, Win_i, bin_i, Wl0_u2i, bl0_u2i, Wr0_u2i, br0_u2i, Wl0_i2u, bl0_i2u, Wr0_i2u, br0_i2u, Wl1_u2i, bl1_u2i, Wr1_u2i, br1_u2i, Wl1_i2u, bl1_i2u, Wr1_i2u, br1_i2u):
    def prep(ei, n_src):
        src = ei[0].reshape(NS, NBLK, KB, EC)
        srcs3 = jnp.stack([src, src + n_src])
        dsts3 = ei[1].reshape(NS, NBLK, KB, EC)
        return srcs3, dsts3

    srcs_u2i, dsts_u2i = prep(ei_u2i, NU)
    srcs_i2u, dsts_i2u = prep(ei_i2u, NI)
    zeros = jnp.zeros((1000, 128), jnp.float32)

    dst_both = jnp.stack([dsts_u2i, dsts_i2u])
    ones_p = jnp.ones((EC, 128), jnp.float32)
    cnts = _counts_call(dst_both, ones_p, zeros)
    inv_i = (1.0 / jnp.maximum(cnts[0, :, 0], 1.0)).reshape(NI, 1)
    inv_u = (1.0 / jnp.maximum(cnts[1, :, 0], 1.0)).reshape(NU, 1)

    hu0, pi0 = _proj_left(x_user, Win_u, bin_u, Wl0_u2i)
    hi0, pu0 = _proj_left(x_item, Win_i, bin_i, Wl0_i2u)
    si0 = _segsum(pi0.reshape(2 * NU, 128), srcs_u2i, dsts_u2i, zeros)
    su0 = _segsum(pu0.reshape(2 * NI, 128), srcs_i2u, dsts_i2u, zeros)
    hi1, pu1 = _epi_left(si0, inv_i, hi0, Wr0_u2i, bl0_u2i + br0_u2i, Wl1_i2u)
    hu1, pi1 = _epi_left(su0, inv_u, hu0, Wr0_i2u, bl0_i2u + br0_i2u, Wl1_u2i)
    si1 = _segsum(pi1.reshape(2 * NU, 128), srcs_u2i, dsts_u2i, zeros)
    su1 = _segsum(pu1.reshape(2 * NI, 128), srcs_i2u, dsts_i2u, zeros)
    hi2 = _epilogue(si1, inv_i, hi1, Wr1_u2i, bl1_u2i + br1_u2i)
    hu2 = _epilogue(su1, inv_u, hu1, Wr1_i2u, bl1_i2u + br1_i2u)
    return hu2, hi2


# speedup vs baseline: 1.2254x; 1.0002x over previous
import jax
import jax.numpy as jnp
from jax import lax
from jax.experimental import pallas as pl
from jax.experimental.pallas import tpu as pltpu
from jax.experimental.pallas import tpu_sc as plsc

NU = 10000
NI = 10000
E = 160000
DIN = 384
H = 256
BM = 1000

NS = 16
NC = 2
EC = 125
NCH = E // (NS * EC)
KB = 16
NBLK = NCH // KB
GRP = 4
ROWS_PER_TILE = NU // NS


def _split(p):
    return jnp.stack([p[:, :128], p[:, 128:]])


def _proj_left_body(x_ref, w_ref, b_ref, wl_ref, h_ref, p_ref):
    h = jnp.dot(x_ref[:], w_ref[:], preferred_element_type=jnp.float32) + b_ref[:]
    h_ref[:] = h
    p_ref[:] = _split(jnp.dot(h, wl_ref[:], preferred_element_type=jnp.float32))


def _proj_left(x, w, b, wl):
    n, k = x.shape
    return pl.pallas_call(
        _proj_left_body,
        grid=(n // BM,),
        in_specs=[
            pl.BlockSpec((BM, k), lambda m: (m, 0)),
            pl.BlockSpec((k, H), lambda m: (0, 0)),
            pl.BlockSpec((1, H), lambda m: (0, 0)),
            pl.BlockSpec((H, H), lambda m: (0, 0)),
        ],
        out_specs=[
            pl.BlockSpec((BM, H), lambda m: (m, 0)),
            pl.BlockSpec((2, BM, 128), lambda m: (0, m, 0)),
        ],
        out_shape=[
            jax.ShapeDtypeStruct((n, H), jnp.float32),
            jax.ShapeDtypeStruct((2, n, 128), jnp.float32),
        ],
    )(x, w, b.reshape(1, H), wl)


def _epi_body(s_ref, inv_ref, h_ref, w_ref, b_ref, o_ref):
    right = jnp.dot(h_ref[:], w_ref[:], preferred_element_type=jnp.float32) + b_ref[:]
    left = jnp.concatenate([s_ref[0], s_ref[1]], axis=1) * inv_ref[:]
    o_ref[:] = jnp.maximum(left + right, 0.0)


def _epi_left_body(s_ref, inv_ref, h_ref, w_ref, b_ref, wl_ref, o_ref, p_ref):
    right = jnp.dot(h_ref[:], w_ref[:], preferred_element_type=jnp.float32) + b_ref[:]
    left = jnp.concatenate([s_ref[0], s_ref[1]], axis=1) * inv_ref[:]
    o = jnp.maximum(left + right, 0.0)
    o_ref[:] = o
    p_ref[:] = _split(jnp.dot(o, wl_ref[:], preferred_element_type=jnp.float32))


def _epi_left(s2, inv, h, wr, b, wl):
    n = h.shape[0]
    return pl.pallas_call(
        _epi_left_body,
        grid=(n // BM,),
        in_specs=[
            pl.BlockSpec((2, BM, 128), lambda m: (0, m, 0)),
            pl.BlockSpec((BM, 1), lambda m: (m, 0)),
            pl.BlockSpec((BM, H), lambda m: (m, 0)),
            pl.BlockSpec((H, H), lambda m: (0, 0)),
            pl.BlockSpec((1, H), lambda m: (0, 0)),
            pl.BlockSpec((H, H), lambda m: (0, 0)),
        ],
        out_specs=[
            pl.BlockSpec((BM, H), lambda m: (m, 0)),
            pl.BlockSpec((2, BM, 128), lambda m: (0, m, 0)),
        ],
        out_shape=[
            jax.ShapeDtypeStruct((n, H), jnp.float32),
            jax.ShapeDtypeStruct((2, n, 128), jnp.float32),
        ],
    )(s2, inv, h, wr, b.reshape(1, H), wl)


def _epilogue(s2, inv, h, wr, b):
    n = h.shape[0]
    return pl.pallas_call(
        _epi_body,
        grid=(n // BM,),
        in_specs=[
            pl.BlockSpec((2, BM, 128), lambda m: (0, m, 0)),
            pl.BlockSpec((BM, 1), lambda m: (m, 0)),
            pl.BlockSpec((BM, H), lambda m: (m, 0)),
            pl.BlockSpec((H, H), lambda m: (0, 0)),
            pl.BlockSpec((1, H), lambda m: (0, 0)),
        ],
        out_specs=pl.BlockSpec((BM, H), lambda m: (m, 0)),
        out_shape=jax.ShapeDtypeStruct((n, H), jnp.float32),
    )(s2, inv, h, wr, b.reshape(1, H))


_MESH = plsc.VectorSubcoreMesh(core_axis_name="c", subcore_axis_name="s")


def _segsum_body(p2, srcs, dsts, zeros, out, acc, srcv, dstv, rows, gsem, ssem):
    c = lax.axis_index("c")
    s = lax.axis_index("s")
    @pl.when(s < 10)
    def _():
        pltpu.sync_copy(zeros, acc.at[pl.ds(s * 1000, 1000)])
    plsc.subcore_barrier()

    def block(kb, carry):
        pltpu.sync_copy(srcs.at[c, s, kb], srcv)
        pltpu.sync_copy(dsts.at[s, kb], dstv)

        def group(g, carry2):
            base = g * GRP
            g0 = pltpu.async_copy(p2.at[srcv.at[base]], rows.at[0], gsem)
            g1 = pltpu.async_copy(p2.at[srcv.at[base + 1]], rows.at[1], gsem)
            g0.wait()
            s0 = pltpu.async_copy(rows.at[0], acc.at[dstv.at[base]],
                                  ssem.at[0], add=True)
            g1.wait()
            s1 = pltpu.async_copy(rows.at[1], acc.at[dstv.at[base + 1]],
                                  ssem.at[1], add=True)
            s0.wait()
            g2 = pltpu.async_copy(p2.at[srcv.at[base + 2]], rows.at[0], gsem)
            s1.wait()
            g3 = pltpu.async_copy(p2.at[srcv.at[base + 3]], rows.at[1], gsem)
            g2.wait()
            s2 = pltpu.async_copy(rows.at[0], acc.at[dstv.at[base + 2]],
                                  ssem.at[0], add=True)
            g3.wait()
            s3 = pltpu.async_copy(rows.at[1], acc.at[dstv.at[base + 3]],
                                  ssem.at[1], add=True)
            s2.wait()
            s3.wait()
            return carry2

        lax.fori_loop(0, KB // GRP, group, 0)
        return carry

    lax.fori_loop(0, NBLK, block, 0)
    plsc.subcore_barrier()
    @pl.when(s < 10)
    def _():
        pltpu.sync_copy(acc.at[pl.ds(s * 1000, 1000)],
                        out.at[pl.ds(c * NU + s * 1000, 1000)])


_segsum_call = pl.kernel(
    _segsum_body,
    out_type=jax.ShapeDtypeStruct((2 * NU, 128), jnp.float32),
    mesh=_MESH,
    scratch_types=[
        pltpu.VMEM_SHARED((NU, 128), jnp.float32),
        pltpu.VMEM((KB, EC), jnp.int32),
        pltpu.VMEM((KB, EC), jnp.int32),
        pltpu.VMEM((2, EC, 128), jnp.float32),
        pltpu.SemaphoreType.DMA,
        pltpu.SemaphoreType.DMA((2,)),
    ],
)


def _segsum(p2, srcs3, dsts3, zeros):
    return _segsum_call(p2, srcs3, dsts3, zeros).reshape(2, NU, 128)


_EPT = E // NS


def _counts_body(dsts, ones_hbm, zeros, out, acc, dstv, ones_buf, csem):
    c = lax.axis_index("c")
    s = lax.axis_index("s")

    @pl.when(s < 10)
    def _():
        pltpu.sync_copy(zeros, acc.at[pl.ds(s * 1000, 1000)])
    pltpu.sync_copy(ones_hbm, ones_buf)
    pltpu.sync_copy(dsts.at[c, s], dstv)
    plsc.subcore_barrier()

    def group(g, carry):
        base = g * GRP
        descs = []
        for i in range(GRP):
            t = base + i
            descs.append(pltpu.async_copy(
                ones_buf, acc.at[dstv.at[t // KB, t % KB]], csem.at[i],
                add=True))
        for d in descs:
            d.wait()
        return carry

    lax.fori_loop(0, NCH // GRP, group, 0)
    plsc.subcore_barrier()
    @pl.when(s < 10)
    def _():
        pltpu.sync_copy(acc.at[pl.ds(s * 1000, 1000)], out.at[c, pl.ds(s * 1000, 1000)])


_counts_call = pl.kernel(
    _counts_body,
    out_type=jax.ShapeDtypeStruct((2, NU, 128), jnp.float32),
    mesh=_MESH,
    scratch_types=[
        pltpu.VMEM_SHARED((NU, 128), jnp.float32),
        pltpu.VMEM((NBLK, KB, EC), jnp.int32),
        pltpu.VMEM((EC, 128), jnp.float32),
        pltpu.SemaphoreType.DMA((GRP,)),
    ],
)


def kernel(x_user, x_item, ei_u2i, ei_i2u, Win_u, bin_u, Win_i, bin_i, Wl0_u2i, bl0_u2i, Wr0_u2i, br0_u2i, Wl0_i2u, bl0_i2u, Wr0_i2u, br0_i2u, Wl1_u2i, bl1_u2i, Wr1_u2i, br1_u2i, Wl1_i2u, bl1_i2u, Wr1_i2u, br1_i2u):
    def prep(ei, n_src):
        src = ei[0].reshape(NS, NBLK, KB, EC)
        srcs3 = jnp.stack([src, src + n_src])
        dsts3 = ei[1].reshape(NS, NBLK, KB, EC)
        return srcs3, dsts3

    srcs_u2i, dsts_u2i = prep(ei_u2i, NU)
    srcs_i2u, dsts_i2u = prep(ei_i2u, NI)
    zeros = jnp.zeros((1000, 128), jnp.float32)

    dst_both = jnp.stack([dsts_u2i, dsts_i2u])
    ones_p = jnp.ones((EC, 128), jnp.float32)
    cnts = _counts_call(dst_both, ones_p, zeros)
    inv_i = (1.0 / jnp.maximum(cnts[0, :, 0], 1.0)).reshape(NI, 1)
    inv_u = (1.0 / jnp.maximum(cnts[1, :, 0], 1.0)).reshape(NU, 1)

    hu0, pi0 = _proj_left(x_user, Win_u, bin_u, Wl0_u2i)
    hi0, pu0 = _proj_left(x_item, Win_i, bin_i, Wl0_i2u)
    si0 = _segsum(pi0.reshape(2 * NU, 128), srcs_u2i, dsts_u2i, zeros)
    su0 = _segsum(pu0.reshape(2 * NI, 128), srcs_i2u, dsts_i2u, zeros)
    hi1, pu1 = _epi_left(si0, inv_i, hi0, Wr0_u2i, bl0_u2i + br0_u2i, Wl1_i2u)
    hu1, pi1 = _epi_left(su0, inv_u, hu0, Wr0_i2u, bl0_i2u + br0_i2u, Wl1_u2i)
    si1 = _segsum(pi1.reshape(2 * NU, 128), srcs_u2i, dsts_u2i, zeros)
    su1 = _segsum(pu1.reshape(2 * NI, 128), srcs_i2u, dsts_i2u, zeros)
    hi2 = _epilogue(si1, inv_i, hi1, Wr1_u2i, bl1_u2i + br1_u2i)
    hu2 = _epilogue(su1, inv_u, hu1, Wr1_i2u, bl1_i2u + br1_i2u)
    return hu2, hi2
